# baseline (speedup 1.0000x reference)
import jax
import jax.numpy as jnp
from jax import lax
from jax.experimental import pallas as pl
from jax.experimental.pallas import tpu as pltpu
from jax.experimental.pallas import tpu_sc as plsc

VOCAB = 100000
EMBED = 128
BATCH = 4096
HIST = 50

NC = 2
NS = 16
NW = NC * NS

B_PER_W = BATCH // NW
CB = 2
CHUNK = CB * HIST
CHUNK_PAD = 104
N_CHUNKS = B_PER_W // CB
NBUF = 8
LAG = 3
N_GROUPS = N_CHUNKS // NBUF


def _gather_body(idx_hbm, table_hbm, out_hbm, idx_v, *bufs_and_sems):
    bufs = bufs_and_sems[:NBUF]
    gsem = bufs_and_sems[NBUF:2 * NBUF]
    osem = bufs_and_sems[2 * NBUF:]
    wid = lax.axis_index("s") * NC + lax.axis_index("c")
    batch_base = wid * B_PER_W

    pltpu.sync_copy(idx_hbm.at[wid], idx_v)

    def start_gather(g, b):
        pltpu.async_copy(table_hbm.at[idx_v.at[g]], bufs[b], gsem[b])

    def wait_gather(g, b):
        pltpu.make_async_copy(
            table_hbm.at[idx_v.at[g]], bufs[b], gsem[b]
        ).wait()

    def _out_refs(g, b):
        src = bufs[b].at[pl.ds(0, CHUNK)].reshape(CB, HIST, EMBED)
        dst = out_hbm.at[pl.ds(batch_base + g * CB, CB)]
        return src, dst

    def start_out(g, b):
        src, dst = _out_refs(g, b)
        pltpu.async_copy(src, dst, osem[b])

    def wait_out(g, b):
        src, dst = _out_refs(g, b)
        pltpu.make_async_copy(src, dst, osem[b]).wait()

    for b in range(NBUF):
        start_gather(b, b)

    def group(t, _):
        for b in range(NBUF):
            g = t * NBUF + b
            wait_gather(g, b)
            start_out(g, b)
            gp = g - LAG
            bp = (b - LAG) % NBUF

            @pl.when(gp >= 0)
            def _(gp=gp, bp=bp):
                wait_out(gp, bp)

            @pl.when(jnp.logical_and(gp >= 0, gp + NBUF < N_CHUNKS))
            def _(gp=gp, bp=bp):
                start_gather(gp + NBUF, bp)

        return 0

    lax.fori_loop(0, N_GROUPS, group, 0)

    for g in range(N_CHUNKS - LAG, N_CHUNKS):
        wait_out(g, g % NBUF)


@jax.jit
def _embedding_gather(idxp, table):
    mesh = plsc.VectorSubcoreMesh(
        core_axis_name="c", subcore_axis_name="s",
        num_cores=NC, num_subcores=NS,
    )
    k = pl.kernel(
        _gather_body,
        out_type=jax.ShapeDtypeStruct((BATCH, HIST, EMBED), jnp.float32),
        mesh=mesh,
        scratch_types=[
            pltpu.VMEM((N_CHUNKS, CHUNK_PAD), jnp.int32),
        ] + [pltpu.VMEM((CHUNK_PAD, EMBED), jnp.float32)] * NBUF
          + [pltpu.SemaphoreType.DMA] * (2 * NBUF),
    )
    return k(idxp, table)


def kernel(indices, table):
    idxp = jnp.pad(
        indices.astype(jnp.int32).reshape(NW, N_CHUNKS, CHUNK),
        ((0, 0), (0, 0), (0, CHUNK_PAD - CHUNK)),
    )
    return _embedding_gather(idxp, table)
